# baseline (speedup 1.0000x reference)
import functools

import jax
import jax.numpy as jnp
from jax import lax
from jax.experimental import pallas as pl
from jax.experimental.pallas import tpu as pltpu
from jax.experimental.pallas import tpu_sc as plsc

_NUM_CORES = 2
_NUM_SUBCORES = 16
_NW = _NUM_CORES * _NUM_SUBCORES
_LANES = 16


def _make_pool_kernel(B, C, V, D):
    b_per_w = B // _NW
    R = 4
    G = R * C
    assert G <= 128
    assert G % 8 == 0
    n_chunks = b_per_w // R
    assert n_chunks % 2 == 0
    inv_c = 1.0 / C

    mesh = plsc.VectorSubcoreMesh(
        core_axis_name="c", subcore_axis_name="s",
        num_cores=_NUM_CORES, num_subcores=_NUM_SUBCORES)

    @functools.partial(
        pl.kernel,
        out_type=jax.ShapeDtypeStruct((B, D), jnp.float32),
        mesh=mesh,
        scratch_types=[
            pltpu.VMEM(((n_chunks + 1) * G,), jnp.int32),
            pltpu.VMEM((G, D), jnp.float32),
            pltpu.VMEM((G, D), jnp.float32),
            pltpu.VMEM((b_per_w, D), jnp.float32),
            pltpu.SemaphoreType.DMA,
            pltpu.SemaphoreType.DMA,
        ],
    )
    def pool(idx_hbm, table_hbm, out_hbm, idx_v, rows0, rows1, acc_v, sem0, sem1):
        wid = lax.axis_index("s") * _NUM_CORES + lax.axis_index("c")
        base = pl.multiple_of(wid * b_per_w, 8)
        pltpu.sync_copy(idx_hbm.at[pl.ds(base * C, b_per_w * C)], idx_v.at[pl.ds(0, b_per_w * C)])
        pltpu.sync_copy(idx_hbm.at[pl.ds(base * C, G)],
                        idx_v.at[pl.ds(n_chunks * G, G)])

        def fire(chunk, buf, sem):
            off = pl.multiple_of(chunk * G, 8)
            return pltpu.async_copy(table_hbm.at[idx_v.at[pl.ds(off, G)]], buf, sem)

        def drain(buf, sem):
            pltpu.make_async_copy(table_hbm.at[pl.ds(0, G)], buf, sem).wait()

        def accumulate(chunk, buf):
            for r in range(R):
                row = chunk * R + r
                for d in range(D // _LANES):
                    sl = pl.ds(d * _LANES, _LANES)
                    acc = buf[r * C, sl]
                    for j in range(1, C):
                        acc = acc + buf[r * C + j, sl]
                    acc_v[row, sl] = acc * inv_c

        fire(0, rows0, sem0)

        def pair_body(k, carry):
            c0 = k * 2
            fire(c0 + 1, rows1, sem1)
            drain(rows0, sem0)
            accumulate(c0, rows0)
            fire(c0 + 2, rows0, sem0)
            drain(rows1, sem1)
            accumulate(c0 + 1, rows1)
            return carry

        lax.fori_loop(0, n_chunks // 2, pair_body, 0)
        drain(rows0, sem0)
        pltpu.sync_copy(acc_v, out_hbm.at[pl.ds(base, b_per_w)])

    return pool


def _projection_half(pooled_h, w_out, b_row, B, half, prev=None):
    B2, D = pooled_h.shape
    V = w_out.shape[0]
    VT = 1024
    grid = (pl.cdiv(V, VT),)

    def body(p_ref, w_ref, b_ref, *rest):
        o_ref = rest[-1]
        bias_col = jnp.transpose(b_ref[...])
        o_ref[...] = lax.dot_general(
            w_ref[...].astype(jnp.bfloat16), p_ref[...].astype(jnp.bfloat16),
            (((1,), (1,)), ((), ())),
            preferred_element_type=jnp.float32) + bias_col

    in_specs = [
        pl.BlockSpec((B2, D), lambda v: (0, 0)),
        pl.BlockSpec((VT, D), lambda v: (v, 0)),
        pl.BlockSpec((1, VT), lambda v: (0, v)),
    ]
    args = [pooled_h, w_out, b_row]
    aliases = {}
    if prev is not None:
        in_specs.append(pl.BlockSpec(memory_space=pltpu.MemorySpace.HBM))
        args.append(prev)
        aliases = {3: 0}

    return pl.pallas_call(
        body,
        grid=grid,
        in_specs=in_specs,
        out_specs=pl.BlockSpec((VT, B2), lambda v, h=half: (v, h)),
        out_shape=jax.ShapeDtypeStruct((V, B), jnp.float32),
        input_output_aliases=aliases,
        compiler_params=pltpu.CompilerParams(
            dimension_semantics=("parallel",),
            vmem_limit_bytes=100 * 1024 * 1024,
        ),
    )(*args)


def kernel(context, embeddings, W_out, b_out):
    B, C = context.shape
    V, D = embeddings.shape
    B2 = B // 2
    idx_flat = context.reshape(-1).astype(jnp.int32)
    pool = _make_pool_kernel(B2, C, V, D)
    pooled0 = pool(idx_flat[: B2 * C], embeddings)
    pooled1 = pool(idx_flat[B2 * C:], embeddings)
    b_row = b_out.reshape(1, V)
    half0 = _projection_half(pooled0, W_out, b_row, B, 0)
    logits_t = _projection_half(pooled1, W_out, b_row, B, 1, prev=half0)
    return logits_t.T


# speedup vs baseline: 1.0259x; 1.0259x over previous
import functools

import jax
import jax.numpy as jnp
from jax import lax
from jax.experimental import pallas as pl
from jax.experimental.pallas import tpu as pltpu
from jax.experimental.pallas import tpu_sc as plsc

_NUM_CORES = 2
_NUM_SUBCORES = 16
_NW = _NUM_CORES * _NUM_SUBCORES
_LANES = 16


def _make_pool_kernel(B, C, V, D):
    b_per_w = B // _NW
    R = 4
    G = R * C
    assert G <= 128
    assert G % 8 == 0
    n_chunks = b_per_w // R
    assert n_chunks % 2 == 0
    inv_c = 1.0 / C

    mesh = plsc.VectorSubcoreMesh(
        core_axis_name="c", subcore_axis_name="s",
        num_cores=_NUM_CORES, num_subcores=_NUM_SUBCORES)

    @functools.partial(
        pl.kernel,
        out_type=jax.ShapeDtypeStruct((B, D), jnp.float32),
        mesh=mesh,
        scratch_types=[
            pltpu.VMEM(((n_chunks + 1) * G,), jnp.int32),
            pltpu.VMEM((G, D), jnp.float32),
            pltpu.VMEM((G, D), jnp.float32),
            pltpu.VMEM((b_per_w, D), jnp.float32),
            pltpu.SemaphoreType.DMA,
            pltpu.SemaphoreType.DMA,
        ],
    )
    def pool(idx_hbm, table_hbm, out_hbm, idx_v, rows0, rows1, acc_v, sem0, sem1):
        wid = lax.axis_index("s") * _NUM_CORES + lax.axis_index("c")
        base = pl.multiple_of(wid * b_per_w, 8)
        pltpu.sync_copy(idx_hbm.at[pl.ds(base * C, b_per_w * C)], idx_v.at[pl.ds(0, b_per_w * C)])
        pltpu.sync_copy(idx_hbm.at[pl.ds(base * C, G)],
                        idx_v.at[pl.ds(n_chunks * G, G)])

        def fire(chunk, buf, sem):
            off = pl.multiple_of(chunk * G, 8)
            return pltpu.async_copy(table_hbm.at[idx_v.at[pl.ds(off, G)]], buf, sem)

        def drain(buf, sem):
            pltpu.make_async_copy(table_hbm.at[pl.ds(0, G)], buf, sem).wait()

        def accumulate(chunk, buf):
            for r in range(R):
                row = chunk * R + r
                for d in range(D // _LANES):
                    sl = pl.ds(d * _LANES, _LANES)
                    acc = buf[r * C, sl]
                    for j in range(1, C):
                        acc = acc + buf[r * C + j, sl]
                    acc_v[row, sl] = acc * inv_c

        fire(0, rows0, sem0)

        def pair_body(k, carry):
            c0 = k * 2
            fire(c0 + 1, rows1, sem1)
            drain(rows0, sem0)
            accumulate(c0, rows0)
            fire(c0 + 2, rows0, sem0)
            drain(rows1, sem1)
            accumulate(c0 + 1, rows1)
            return carry

        lax.fori_loop(0, n_chunks // 2, pair_body, 0)
        drain(rows0, sem0)
        pltpu.sync_copy(acc_v, out_hbm.at[pl.ds(base, b_per_w)])

    return pool


def _projection(pooled, w_out, b_row):
    B, D = pooled.shape
    V = w_out.shape[0]
    VT = 1024
    grid = (pl.cdiv(V, VT),)

    def body(p_ref, w_ref, b_ref, o_ref):
        bias_col = jnp.transpose(b_ref[...])
        o_ref[...] = lax.dot_general(
            w_ref[...].astype(jnp.bfloat16), p_ref[...].astype(jnp.bfloat16),
            (((1,), (1,)), ((), ())),
            preferred_element_type=jnp.float32) + bias_col

    return pl.pallas_call(
        body,
        grid=grid,
        in_specs=[
            pl.BlockSpec((B, D), lambda v: (0, 0)),
            pl.BlockSpec((VT, D), lambda v: (v, 0)),
            pl.BlockSpec((1, VT), lambda v: (0, v)),
        ],
        out_specs=pl.BlockSpec((VT, B), lambda v: (v, 0)),
        out_shape=jax.ShapeDtypeStruct((V, B), jnp.float32),
        compiler_params=pltpu.CompilerParams(
            dimension_semantics=("parallel",),
            vmem_limit_bytes=100 * 1024 * 1024,
        ),
    )(pooled, w_out, b_row)


def kernel(context, embeddings, W_out, b_out):
    B, C = context.shape
    V, D = embeddings.shape
    idx_flat = context.reshape(-1).astype(jnp.int32)
    pool = _make_pool_kernel(B, C, V, D)
    pooled = pool(idx_flat, embeddings)
    logits_t = _projection(pooled, W_out, b_out.reshape(1, V))
    return logits_t.T
